# baseline (speedup 1.0000x reference)
import functools

import jax
import jax.numpy as jnp
from jax import lax
from jax.experimental import pallas as pl
from jax.experimental.pallas import tpu as pltpu
from jax.experimental.pallas import tpu_sc as plsc

N_FACTORS = 64
HIDDEN_1 = 256
BATCH = 16384

_NC = 2
_NS = 16
_NW = _NC * _NS
_BPW = BATCH // _NW
_CHUNK = 128
_NCHUNK = _BPW // _CHUNK


def _gather_body(uid_hbm, iid_hbm, utab_hbm, itab_hbm, u_out, i_out,
                 idx_u, idx_i, rows_u, rows_i, sem_u, sem_i):
    wid = lax.axis_index("s") * _NC + lax.axis_index("c")
    base = wid * _BPW
    pltpu.sync_copy(uid_hbm.at[pl.ds(base, _BPW)], idx_u)
    pltpu.sync_copy(iid_hbm.at[pl.ds(base, _BPW)], idx_i)
    copies = []
    for j in range(_NCHUNK):
        sl = pl.ds(j * _CHUNK, _CHUNK)
        copies.append(pltpu.async_copy(
            utab_hbm.at[idx_u.at[sl]], rows_u.at[sl], sem_u))
        copies.append(pltpu.async_copy(
            itab_hbm.at[idx_i.at[sl]], rows_i.at[sl], sem_i))
    for c in copies:
        c.wait()
    pltpu.sync_copy(rows_u, u_out.at[pl.ds(base, _BPW)])
    pltpu.sync_copy(rows_i, i_out.at[pl.ds(base, _BPW)])


@jax.jit
def _sc_gather(user_ids, item_ids, user_table, item_table):
    mesh = plsc.VectorSubcoreMesh(core_axis_name="c", subcore_axis_name="s")
    f = pl.kernel(
        _gather_body,
        out_type=(
            jax.ShapeDtypeStruct((BATCH, N_FACTORS), jnp.float32),
            jax.ShapeDtypeStruct((BATCH, N_FACTORS), jnp.float32),
        ),
        mesh=mesh,
        scratch_types=[
            pltpu.VMEM((_BPW,), jnp.int32),
            pltpu.VMEM((_BPW,), jnp.int32),
            pltpu.VMEM((_BPW, N_FACTORS), jnp.float32),
            pltpu.VMEM((_BPW, N_FACTORS), jnp.float32),
            pltpu.SemaphoreType.DMA,
            pltpu.SemaphoreType.DMA,
        ],
        compiler_params=pltpu.CompilerParams(use_tc_tiling_on_sc=False),
    )
    return f(user_ids, item_ids, user_table, item_table)


def _mlp_body(u_ref, i_ref, w1u_ref, w1i_ref, b1_ref, w2_ref, b2_ref, o_ref):
    x = jnp.dot(u_ref[...], w1u_ref[...], preferred_element_type=jnp.float32)
    x = x + jnp.dot(i_ref[...], w1i_ref[...], preferred_element_type=jnp.float32)
    h = jnp.maximum(x + b1_ref[...], 0.0)
    o_ref[...] = jnp.dot(h, w2_ref[...], preferred_element_type=jnp.float32) + b2_ref[...]


def _tc_mlp(u_emb, i_emb, w1u_t, w1i_t, b1_2d, w2_c, b2_2d, blk=2048):
    grid = (BATCH // blk,)
    return pl.pallas_call(
        _mlp_body,
        grid=grid,
        in_specs=[
            pl.BlockSpec((blk, N_FACTORS), lambda i: (i, 0)),
            pl.BlockSpec((blk, N_FACTORS), lambda i: (i, 0)),
            pl.BlockSpec((N_FACTORS, HIDDEN_1), lambda i: (0, 0)),
            pl.BlockSpec((N_FACTORS, HIDDEN_1), lambda i: (0, 0)),
            pl.BlockSpec((1, HIDDEN_1), lambda i: (0, 0)),
            pl.BlockSpec((HIDDEN_1, 1), lambda i: (0, 0)),
            pl.BlockSpec((1, 1), lambda i: (0, 0)),
        ],
        out_specs=pl.BlockSpec((blk, 1), lambda i: (i, 0)),
        out_shape=jax.ShapeDtypeStruct((BATCH, 1), jnp.float32),
    )(u_emb, i_emb, w1u_t, w1i_t, b1_2d, w2_c, b2_2d)


def kernel(user_ids, item_ids, user_table, item_table, W1, b1, W2, b2):
    uid = user_ids.astype(jnp.int32)
    iid = item_ids.astype(jnp.int32)
    u_emb, i_emb = _sc_gather(uid, iid, user_table, item_table)
    w1u_t = W1[:, :N_FACTORS].T
    w1i_t = W1[:, N_FACTORS:].T
    out = _tc_mlp(u_emb, i_emb, w1u_t, w1i_t, b1[None, :], W2.T, b2[None, :])
    return out[:, 0]
